# baseline (speedup 1.0000x reference)
import functools

import jax
import jax.numpy as jnp
from jax.experimental import pallas as pl
from jax.experimental.pallas import tpu as pltpu


def _mm_kernel(x_ref, w_ref, b_ref, o_ref):
    o_ref[...] = (
        jnp.dot(x_ref[...], w_ref[...], preferred_element_type=jnp.float32)
        + b_ref[...]
    )


def _mm(x, w, b, bm=512, bn=1024):
    m, k = x.shape
    k2, n = w.shape
    grid = (m // bm, n // bn)
    return pl.pallas_call(
        _mm_kernel,
        grid=grid,
        in_specs=[
            pl.BlockSpec((bm, k), lambda i, j: (i, 0)),
            pl.BlockSpec((k, bn), lambda i, j: (0, j)),
            pl.BlockSpec((bn,), lambda i, j: (j,)),
        ],
        out_specs=pl.BlockSpec((bm, bn), lambda i, j: (i, j)),
        out_shape=jax.ShapeDtypeStruct((m, n), jnp.float32),
    )(x, w, b)


def kernel(q, k, edge_index, edge_features, Wq, bq, Wk, bk, Wv, bv,
           Wqr, bqr, Wkr, bkr, Wvr, bvr, Wo, bo):
    B, L, D = q.shape
    H = 16
    DH = D // H
    N = B * L
    E = edge_index.shape[1]
    scale = 1.0 / jnp.sqrt(jnp.asarray(DH, dtype=jnp.float32))

    qf2 = q.reshape(-1, D)
    kf2 = k.reshape(-1, D)
    Qf = _mm(qf2, Wq, bq) * scale
    Qr = _mm(qf2, Wqr, bqr) * scale
    Kf = _mm(kf2, Wk, bk)
    Vf = _mm(kf2, Wv, bv)
    Kr = _mm(edge_features, Wkr, bkr)
    Vr = _mm(edge_features, Wvr, bvr)

    src = edge_index[0].astype(jnp.int32)
    dst = edge_index[1].astype(jnp.int32)

    qe = jnp.take(Qf, dst, axis=0).reshape(-1, H, DH)
    qre = jnp.take(Qr, dst, axis=0).reshape(-1, H, DH)
    ke = jnp.take(Kf, src, axis=0).reshape(-1, H, DH)
    ve = jnp.take(Vf, src, axis=0).reshape(-1, H, DH)
    kre = Kr.reshape(-1, H, DH)
    vre = Vr.reshape(-1, H, DH)

    score = (qe * ke).sum(-1) + (qre * kre).sum(-1)
    w = jnp.exp(score)
    denom = jax.ops.segment_sum(w, dst, num_segments=N)
    U = jax.ops.segment_sum(w[:, :, None] * (ve + vre), dst, num_segments=N)
    agg = U / (denom[:, :, None] + 1e-9)
    out = _mm(agg.reshape(N, D), Wo, bo)
    return out.reshape(B, L, D)
